# Initial kernel scaffold; baseline (speedup 1.0000x reference)
import functools

import jax
import jax.numpy as jnp
from jax import lax
from jax.experimental import pallas as pl
from jax.experimental.pallas import tpu as pltpu
from jax.experimental.pallas import tpu_sc as plsc

KCAP = 4
DD = 32
D = 128
NB = 128
C = 2048
BIGC = 4
BIG = BIGC * C
ROUTE_ITERS = 6
SC_G = 512
SC_WORKERS = 32


def _sel(i, j):
    return (i == j).astype(jnp.float32)


def _mk_consts():
    w1 = _sel(lax.broadcasted_iota(jnp.int32, (D, D), 0) // KCAP,
              lax.broadcasted_iota(jnp.int32, (D, D), 1) % DD)
    g = _sel(lax.broadcasted_iota(jnp.int32, (D, KCAP), 0) // DD,
             lax.broadcasted_iota(jnp.int32, (D, KCAP), 1))
    gt = _sel(lax.broadcasted_iota(jnp.int32, (KCAP, D), 0),
              lax.broadcasted_iota(jnp.int32, (KCAP, D), 1) // DD)
    return w1, g, gt


def _group_normalize(v, g, gt):
    nrm2 = jnp.dot(v * v, g, preferred_element_type=jnp.float32)
    inv = 1.0 / jnp.maximum(jnp.sqrt(nrm2), 1e-12)
    return v * jnp.dot(inv, gt, preferred_element_type=jnp.float32)


def _norm_body(x_ref, o_ref):
    _, g, gt = _mk_consts()
    o_ref[...] = _group_normalize(x_ref[...], g, gt)


def _normalize(x_pad):
    return pl.pallas_call(
        _norm_body,
        out_shape=jax.ShapeDtypeStruct(x_pad.shape, jnp.float32),
    )(x_pad)


def _gather_rows(table, idx):
    mpad = idx.shape[0]
    per_w = mpad // SC_WORKERS
    nchunk = per_w // SC_G
    mesh = plsc.VectorSubcoreMesh(core_axis_name="c", subcore_axis_name="s")

    @functools.partial(
        pl.kernel,
        mesh=mesh,
        out_type=jax.ShapeDtypeStruct((mpad, D), jnp.float32),
        scratch_types=[
            pltpu.VMEM((SC_G,), jnp.int32),
            pltpu.VMEM((SC_G, D), jnp.float32),
            pltpu.SemaphoreType.DMA,
        ],
    )
    def sc_gather(table_hbm, idx_hbm, out_hbm, idx_v, rows_v, sem):
        wid = lax.axis_index("s") * 2 + lax.axis_index("c")
        base = wid * per_w

        @pl.loop(0, nchunk)
        def _(i):
            off = base + i * SC_G
            pltpu.sync_copy(idx_hbm.at[pl.ds(off, SC_G)], idx_v)
            pltpu.async_copy(table_hbm.at[idx_v], rows_v, sem).wait()
            pltpu.sync_copy(rows_v, out_hbm.at[pl.ds(off, SC_G)])

    return sc_gather(table, idx)


def _routing_body(starts_ref, xn_ref, z_hbm, trg_hbm, out_ref,
                  z_buf, trg_buf, dma_sem):
    b = pl.program_id(0)
    start = starts_ref[b]
    n_e = starts_ref[b + 1] - start
    nch = lax.div(n_e + (C - 1), C)
    fits = n_e <= BIG

    w1, g, gt = _mk_consts()
    base_n = b * NB
    iota_n = lax.broadcasted_iota(jnp.int32, (NB, C), 0) + base_n

    x_blk = xn_ref[...]
    c = x_blk

    for t in range(ROUTE_ITERS):
        cst = jnp.dot(c, w1, preferred_element_type=jnp.float32)

        def chunk_body(i, acc, do_dma):
            slot = jnp.where(fits, i, 0)
            if do_dma:
                cp_z = pltpu.make_async_copy(
                    z_hbm.at[pl.ds(start + i * C, C), :],
                    z_buf.at[slot], dma_sem)
                cp_t = pltpu.make_async_copy(
                    trg_hbm.at[:, pl.ds(start + i * C, C)],
                    trg_buf.at[slot], dma_sem)
                cp_z.start()
                cp_t.start()
                cp_z.wait()
                cp_t.wait()
            z_c = z_buf[slot]
            trg_c = trg_buf[slot]
            ot = _sel(iota_n, trg_c)
            csg = lax.dot_general(ot, cst, (((0,), (0,)), ((), ())),
                                  preferred_element_type=jnp.float32)
            prod = z_c * csg
            p4 = jnp.dot(prod, g, preferred_element_type=jnp.float32)
            e4 = jnp.exp(p4)
            p4n = e4 / jnp.sum(e4, axis=1, keepdims=True)
            pt = jnp.dot(p4n, gt, preferred_element_type=jnp.float32)
            w = pt * z_c
            return acc + jnp.dot(ot, w, preferred_element_type=jnp.float32)

        if t == 0:
            body = lambda i, acc: chunk_body(i, acc, True)
        else:
            def body(i, acc):
                return lax.cond(fits,
                                lambda: chunk_body(i, acc, False),
                                lambda: chunk_body(i, acc, True))
        acc = lax.fori_loop(0, nch, body, jnp.zeros((NB, D), jnp.float32))
        c = x_blk + acc
        if t < ROUTE_ITERS - 1:
            c = _group_normalize(c, g, gt)

    out_ref[...] = c


def _routing(starts, xn_pad, z, trg_pad2d, npad):
    grid_spec = pltpu.PrefetchScalarGridSpec(
        num_scalar_prefetch=1,
        grid=(npad // NB,),
        in_specs=[
            pl.BlockSpec((NB, D), lambda b, s: (b, 0)),
            pl.BlockSpec(memory_space=pltpu.ANY),
            pl.BlockSpec(memory_space=pltpu.ANY),
        ],
        out_specs=pl.BlockSpec((NB, D), lambda b, s: (b, 0)),
        scratch_shapes=[
            pltpu.VMEM((BIGC, C, D), jnp.float32),
            pltpu.VMEM((BIGC, 1, C), jnp.int32),
            pltpu.SemaphoreType.DMA,
        ],
    )
    return pl.pallas_call(
        _routing_body,
        grid_spec=grid_spec,
        out_shape=jax.ShapeDtypeStruct((npad, D), jnp.float32),
    )(starts, xn_pad, z, trg_pad2d)


def kernel(x, edge_index):
    n, d = x.shape
    assert d == D
    src = edge_index[0]
    trg = edge_index[1]
    m = src.shape[0]

    nblk = -(-n // NB)
    npad = nblk * NB
    sc_quant = SC_WORKERS * SC_G
    mpad = -(-(m + BIG) // sc_quant) * sc_quant

    order_iota = jnp.arange(m, dtype=jnp.int32)
    trg_s, order = lax.sort_key_val(trg, order_iota, is_stable=False)
    src_s = jnp.take(src, order)
    bounds = (jnp.arange(nblk + 1, dtype=jnp.int32) * NB)
    starts = jnp.searchsorted(trg_s, bounds, side="left").astype(jnp.int32)

    pad_m = mpad - m
    src_pad = jnp.concatenate(
        [src_s, (jnp.arange(pad_m, dtype=jnp.int32) % n)])
    trg_pad = jnp.concatenate(
        [trg_s, jnp.full((pad_m,), npad, jnp.int32)]).reshape(1, mpad)

    x_pad = jnp.pad(x, ((0, npad - n), (0, 0)))
    xn_pad = _normalize(x_pad)
    z = _gather_rows(xn_pad, src_pad)
    c = _routing(starts, xn_pad, z, trg_pad, npad)
    return c[:n]


# speedup vs baseline: 8.5142x; 8.5142x over previous
import functools

import jax
import jax.numpy as jnp
from jax import lax
from jax.experimental import pallas as pl
from jax.experimental.pallas import tpu as pltpu
from jax.experimental.pallas import tpu_sc as plsc

KCAP = 4
DD = 32
D = 128
NB = 128
C = 2048
BIGC = 4
BIG = BIGC * C
ROUTE_ITERS = 6
SC_G = 512
SC_WORKERS = 32


def _sel(i, j):
    return (i == j).astype(jnp.float32)


def _mk_consts():
    w1 = _sel(lax.broadcasted_iota(jnp.int32, (D, D), 0) // KCAP,
              lax.broadcasted_iota(jnp.int32, (D, D), 1) % DD)
    g = _sel(lax.broadcasted_iota(jnp.int32, (D, KCAP), 0) // DD,
             lax.broadcasted_iota(jnp.int32, (D, KCAP), 1))
    gt = _sel(lax.broadcasted_iota(jnp.int32, (KCAP, D), 0),
              lax.broadcasted_iota(jnp.int32, (KCAP, D), 1) // DD)
    return w1, g, gt


def _group_normalize(v, g, gt):
    nrm2 = jnp.dot(v * v, g, preferred_element_type=jnp.float32)
    inv = 1.0 / jnp.maximum(jnp.sqrt(nrm2), 1e-12)
    return v * jnp.dot(inv, gt, preferred_element_type=jnp.float32)


def _norm_body(x_ref, o_ref):
    _, g, gt = _mk_consts()
    o_ref[...] = _group_normalize(x_ref[...], g, gt)


def _normalize(x_pad):
    return pl.pallas_call(
        _norm_body,
        out_shape=jax.ShapeDtypeStruct(x_pad.shape, jnp.float32),
    )(x_pad)


def _gather_rows(table, idx):
    mpad = idx.shape[0]
    per_w = mpad // SC_WORKERS
    nchunk = per_w // SC_G
    mesh = plsc.VectorSubcoreMesh(core_axis_name="c", subcore_axis_name="s")

    @functools.partial(
        pl.kernel,
        mesh=mesh,
        out_type=jax.ShapeDtypeStruct((mpad, D), jnp.float32),
        scratch_types=[
            pltpu.VMEM((SC_G,), jnp.int32),
            pltpu.VMEM((SC_G, D), jnp.float32),
            pltpu.SemaphoreType.DMA,
        ],
    )
    def sc_gather(table_hbm, idx_hbm, out_hbm, idx_v, rows_v, sem):
        wid = lax.axis_index("s") * 2 + lax.axis_index("c")
        base = wid * per_w

        @pl.loop(0, nchunk)
        def _(i):
            off = base + i * SC_G
            pltpu.sync_copy(idx_hbm.at[pl.ds(off, SC_G)], idx_v)
            pltpu.async_copy(table_hbm.at[idx_v], rows_v, sem).wait()
            pltpu.sync_copy(rows_v, out_hbm.at[pl.ds(off, SC_G)])

    return sc_gather(table, idx)


def _routing_body(starts_ref, xn_ref, z_hbm, trg_hbm, out_ref,
                  z_buf, trg_buf, dma_sem):
    b = pl.program_id(0)
    start = pl.multiple_of((starts_ref[b] // NB) * NB, NB)
    n_e = starts_ref[b + 1] - start
    nch = lax.div(n_e + (C - 1), C)
    fits = n_e <= BIG

    w1, g, gt = _mk_consts()
    base_n = b * NB
    iota_n = lax.broadcasted_iota(jnp.int32, (NB, C), 0) + base_n

    x_blk = xn_ref[...]
    c = x_blk

    for t in range(ROUTE_ITERS):
        cst = jnp.dot(c, w1, preferred_element_type=jnp.float32)

        def chunk_body(i, acc, do_dma):
            slot = jnp.where(fits, i, 0)
            if do_dma:
                cp_z = pltpu.make_async_copy(
                    z_hbm.at[pl.ds(start + i * C, C), :],
                    z_buf.at[slot], dma_sem)
                cp_t = pltpu.make_async_copy(
                    trg_hbm.at[:, pl.ds(start + i * C, C)],
                    trg_buf.at[slot], dma_sem)
                cp_z.start()
                cp_t.start()
                cp_z.wait()
                cp_t.wait()
            z_c = z_buf[slot]
            trg_c = trg_buf[slot]
            ot = _sel(iota_n, trg_c)
            csg = lax.dot_general(ot, cst, (((0,), (0,)), ((), ())),
                                  preferred_element_type=jnp.float32)
            prod = z_c * csg
            p4 = jnp.dot(prod, g, preferred_element_type=jnp.float32)
            e4 = jnp.exp(p4)
            p4n = e4 / jnp.sum(e4, axis=1, keepdims=True)
            pt = jnp.dot(p4n, gt, preferred_element_type=jnp.float32)
            w = pt * z_c
            return acc + jnp.dot(ot, w, preferred_element_type=jnp.float32)

        if t == 0:
            body = lambda i, acc: chunk_body(i, acc, True)
        else:
            def body(i, acc):
                return lax.cond(fits,
                                lambda: chunk_body(i, acc, False),
                                lambda: chunk_body(i, acc, True))
        acc = lax.fori_loop(0, nch, body, jnp.zeros((NB, D), jnp.float32))
        c = x_blk + acc
        if t < ROUTE_ITERS - 1:
            c = _group_normalize(c, g, gt)

    out_ref[...] = c


def _routing(starts, xn_pad, z, trg_pad2d, npad):
    grid_spec = pltpu.PrefetchScalarGridSpec(
        num_scalar_prefetch=1,
        grid=(npad // NB,),
        in_specs=[
            pl.BlockSpec((NB, D), lambda b, s: (b, 0)),
            pl.BlockSpec(memory_space=pl.ANY),
            pl.BlockSpec(memory_space=pl.ANY),
        ],
        out_specs=pl.BlockSpec((NB, D), lambda b, s: (b, 0)),
        scratch_shapes=[
            pltpu.VMEM((BIGC, C, D), jnp.float32),
            pltpu.VMEM((BIGC, 1, C), jnp.int32),
            pltpu.SemaphoreType.DMA,
        ],
    )
    return pl.pallas_call(
        _routing_body,
        grid_spec=grid_spec,
        out_shape=jax.ShapeDtypeStruct((npad, D), jnp.float32),
    )(starts, xn_pad, z, trg_pad2d)


def kernel(x, edge_index):
    n, d = x.shape
    assert d == D
    src = edge_index[0]
    trg = edge_index[1]
    m = src.shape[0]

    nblk = -(-n // NB)
    npad = nblk * NB
    sc_quant = SC_WORKERS * SC_G
    mpad = -(-(m + BIG) // sc_quant) * sc_quant

    order_iota = jnp.arange(m, dtype=jnp.int32)
    trg_s, order = lax.sort_key_val(trg, order_iota, is_stable=False)
    src_s = jnp.take(src, order)
    bounds = (jnp.arange(nblk + 1, dtype=jnp.int32) * NB)
    starts = jnp.searchsorted(trg_s, bounds, side="left").astype(jnp.int32)

    pad_m = mpad - m
    src_pad = jnp.concatenate(
        [src_s, (jnp.arange(pad_m, dtype=jnp.int32) % n)])
    trg_pad = jnp.concatenate(
        [trg_s, jnp.full((pad_m,), npad, jnp.int32)]).reshape(1, mpad)

    x_pad = jnp.pad(x, ((0, npad - n), (0, 0)))
    xn_pad = _normalize(x_pad)
    z = _gather_rows(xn_pad, src_pad)
    c = _routing(starts, xn_pad, z, trg_pad, npad)
    return c[:n]


# speedup vs baseline: 8.5903x; 1.0089x over previous
import functools

import jax
import jax.numpy as jnp
from jax import lax
from jax.experimental import pallas as pl
from jax.experimental.pallas import tpu as pltpu
from jax.experimental.pallas import tpu_sc as plsc

KCAP = 4
DD = 32
D = 128
NB = 128
C = 2048
BIGC = 4
BIG = BIGC * C
ROUTE_ITERS = 6
SC_G = 512
SC_WORKERS = 32


def _sel(i, j):
    return (i == j).astype(jnp.float32)


def _mk_consts():
    w1 = _sel(lax.broadcasted_iota(jnp.int32, (D, D), 0) // KCAP,
              lax.broadcasted_iota(jnp.int32, (D, D), 1) % DD)
    g = _sel(lax.broadcasted_iota(jnp.int32, (D, KCAP), 0) // DD,
             lax.broadcasted_iota(jnp.int32, (D, KCAP), 1))
    gt = _sel(lax.broadcasted_iota(jnp.int32, (KCAP, D), 0),
              lax.broadcasted_iota(jnp.int32, (KCAP, D), 1) // DD)
    return w1, g, gt


def _group_normalize(v, g, gt):
    nrm2 = jnp.dot(v * v, g, preferred_element_type=jnp.float32)
    inv = 1.0 / jnp.maximum(jnp.sqrt(nrm2), 1e-12)
    return v * jnp.dot(inv, gt, preferred_element_type=jnp.float32)


def _norm_body(x_ref, o_ref):
    _, g, gt = _mk_consts()
    o_ref[...] = _group_normalize(x_ref[...], g, gt)


def _normalize(x_pad):
    return pl.pallas_call(
        _norm_body,
        out_shape=jax.ShapeDtypeStruct(x_pad.shape, jnp.float32),
    )(x_pad)


def _gather_rows(table, idx):
    mpad = idx.shape[0]
    per_w = mpad // SC_WORKERS
    nchunk = per_w // SC_G
    mesh = plsc.VectorSubcoreMesh(core_axis_name="c", subcore_axis_name="s")

    @functools.partial(
        pl.kernel,
        mesh=mesh,
        out_type=jax.ShapeDtypeStruct((mpad, D), jnp.float32),
        scratch_types=[
            pltpu.VMEM((SC_G,), jnp.int32),
            pltpu.VMEM((SC_G, D), jnp.float32),
            pltpu.SemaphoreType.DMA,
        ],
    )
    def sc_gather(table_hbm, idx_hbm, out_hbm, idx_v, rows_v, sem):
        wid = lax.axis_index("s") * 2 + lax.axis_index("c")
        base = wid * per_w

        @pl.loop(0, nchunk)
        def _(i):
            off = base + i * SC_G
            pltpu.sync_copy(idx_hbm.at[pl.ds(off, SC_G)], idx_v)
            pltpu.async_copy(table_hbm.at[idx_v], rows_v, sem).wait()
            pltpu.sync_copy(rows_v, out_hbm.at[pl.ds(off, SC_G)])

    return sc_gather(table, idx)


def _routing_body(starts_ref, xn_ref, z_hbm, trg_hbm, out_ref,
                  z_buf, trg_buf, dma_sem):
    b = pl.program_id(0)
    start = pl.multiple_of((starts_ref[b] // NB) * NB, NB)
    n_e = starts_ref[b + 1] - start
    nch = lax.div(n_e + (C - 1), C)
    npair = lax.div(nch + 1, 2)
    fits = n_e <= BIG

    w1, g, gt = _mk_consts()
    base_n = b * NB
    iota_n = lax.broadcasted_iota(jnp.int32, (NB, C), 0) + base_n

    x_blk = xn_ref[...]
    c = x_blk

    for t in range(ROUTE_ITERS):
        cst = jnp.dot(c, w1, preferred_element_type=jnp.float32)
        cst16 = cst.astype(jnp.bfloat16)

        def start_dma(i, slot):
            cp_z = pltpu.make_async_copy(
                z_hbm.at[pl.ds(start + i * C, C), :],
                z_buf.at[slot], dma_sem)
            cp_t = pltpu.make_async_copy(
                trg_hbm.at[:, pl.ds(start + i * C, C)],
                trg_buf.at[slot], dma_sem)
            cp_z.start()
            cp_t.start()
            return cp_z, cp_t

        def chunk_math(slot, acc):
            z_c = z_buf[slot]
            trg_c = trg_buf[slot]
            ot = _sel(iota_n, trg_c)
            ot16 = ot.astype(jnp.bfloat16)
            csg = lax.dot_general(ot16, cst16, (((0,), (0,)), ((), ())),
                                  preferred_element_type=jnp.float32)
            prod = z_c * csg
            p4 = jnp.dot(prod, g, preferred_element_type=jnp.float32)
            e4 = jnp.exp(p4)
            p4n = e4 / jnp.sum(e4, axis=1, keepdims=True)
            pt = jnp.dot(p4n, gt, preferred_element_type=jnp.float32)
            w = (pt * z_c).astype(jnp.bfloat16)
            return acc + jnp.dot(ot16, w, preferred_element_type=jnp.float32)

        def pair_body(j, acc, do_dma):
            i0 = 2 * j
            s0 = jnp.where(fits, i0, 0)
            s1 = jnp.where(fits, i0 + 1, 1)
            if do_dma:
                cps = start_dma(i0, s0) + start_dma(i0 + 1, s1)
                for cp in cps:
                    cp.wait()
            acc = chunk_math(s0, acc)
            return chunk_math(s1, acc)

        if t == 0:
            body = lambda j, acc: pair_body(j, acc, True)
        else:
            def body(j, acc):
                return lax.cond(fits,
                                lambda: pair_body(j, acc, False),
                                lambda: pair_body(j, acc, True))
        acc = lax.fori_loop(0, npair, body, jnp.zeros((NB, D), jnp.float32))
        c = x_blk + acc
        if t < ROUTE_ITERS - 1:
            c = _group_normalize(c, g, gt)

    out_ref[...] = c


def _routing(starts, xn_pad, z, trg_pad2d, npad):
    grid_spec = pltpu.PrefetchScalarGridSpec(
        num_scalar_prefetch=1,
        grid=(npad // NB,),
        in_specs=[
            pl.BlockSpec((NB, D), lambda b, s: (b, 0)),
            pl.BlockSpec(memory_space=pl.ANY),
            pl.BlockSpec(memory_space=pl.ANY),
        ],
        out_specs=pl.BlockSpec((NB, D), lambda b, s: (b, 0)),
        scratch_shapes=[
            pltpu.VMEM((BIGC, C, D), jnp.float32),
            pltpu.VMEM((BIGC, 1, C), jnp.int32),
            pltpu.SemaphoreType.DMA,
        ],
    )
    return pl.pallas_call(
        _routing_body,
        grid_spec=grid_spec,
        out_shape=jax.ShapeDtypeStruct((npad, D), jnp.float32),
    )(starts, xn_pad, z, trg_pad2d)


def kernel(x, edge_index):
    n, d = x.shape
    assert d == D
    src = edge_index[0]
    trg = edge_index[1]
    m = src.shape[0]

    nblk = -(-n // NB)
    npad = nblk * NB
    sc_quant = SC_WORKERS * SC_G
    mpad = -(-(m + BIG) // sc_quant) * sc_quant

    trg_s, src_s = lax.sort((trg, src), num_keys=1, is_stable=False)
    bounds = (jnp.arange(nblk + 1, dtype=jnp.int32) * NB)
    starts = jnp.searchsorted(trg_s, bounds, side="left").astype(jnp.int32)

    pad_m = mpad - m
    src_pad = jnp.concatenate(
        [src_s, (jnp.arange(pad_m, dtype=jnp.int32) % n)])
    trg_pad = jnp.concatenate(
        [trg_s, jnp.full((pad_m,), npad, jnp.int32)]).reshape(1, mpad)

    x_pad = jnp.pad(x, ((0, npad - n), (0, 0)))
    xn_pad = _normalize(x_pad)
    z = _gather_rows(xn_pad, src_pad)
    c = _routing(starts, xn_pad, z, trg_pad, npad)
    return c[:n]
